# baseline (speedup 1.0000x reference)
import jax
import jax.numpy as jnp
from jax.experimental import pallas as pl
from jax.experimental.pallas import tpu as pltpu

N = 1140
H = 600
OUT = 300


def _dot(a, b):
    return jax.lax.dot(a, b, preferred_element_type=jnp.float32)


def _bf(a):
    return a.astype(jnp.bfloat16)


def _gcn_body(x_hbm, A_hbm, W1_hbm, b1_ref, W2_hbm, b2_ref,
              W3a_hbm, W3b_hbm, b3_ref, W4_hbm, b4_ref, W5_hbm, b5_ref,
              xm_ref, out2_ref,
              x_v, A_v, W1_v, W2_v, W3a_v, W3b_v, W4_v, W5_v, sems):
    copies = []
    for i, (src, dst) in enumerate((
            (x_hbm, x_v), (W1_hbm, W1_v), (A_hbm, A_v), (W2_hbm, W2_v),
            (W3a_hbm, W3a_v), (W3b_hbm, W3b_v), (W4_hbm, W4_v),
            (W5_hbm, W5_v))):
        cp = pltpu.make_async_copy(src, dst, sems.at[i])
        cp.start()
        copies.append(cp)
    c_x, c_W1, c_A, c_W2, c_W3a, c_W3b, c_W4, c_W5 = copies

    c_x.wait()
    c_W1.wait()
    y = _bf(_dot(_bf(x_v[...]), _bf(W1_v[...])))

    c_A.wait()
    A = _bf(A_v[...])
    A2 = _bf(_dot(A, A))
    s = jnp.sum(A_v[...], axis=1, keepdims=True)
    c_W2.wait()
    c_W3a.wait()
    W3a = _bf(W3a_v[...])
    W23a = _bf(_dot(_bf(W2_v[...]), W3a))
    c3 = _dot(_bf(b2_ref[...]), W3a) + b3_ref[...]
    c_W4.wait()
    c_W5.wait()
    W5b = _bf(W5_v[...])
    W45 = _bf(_dot(_bf(W4_v[...]), W5b))
    b4W5 = _dot(_bf(b4_ref[...]), W5b)

    x1 = _bf(jnp.maximum(_dot(A, y) + b1_ref[...], 0.0))
    u = _bf(_dot(x1, W23a))
    c_W3b.wait()
    xm = _dot(A, u) + _dot(x1, _bf(W3b_v[...])) + c3
    xm_ref[...] = xm
    P2 = _bf(_dot(A2, _bf(xm)))
    out2_ref[...] = jax.nn.sigmoid(_dot(P2, W45) + s * b4W5 + b5_ref[...])


def kernel(x, A, W1, b1, W2, b2, W3, b3, W4, b4, W5, b5):
    args = (
        x, A,
        W1, b1.reshape(1, H),
        W2, b2.reshape(1, OUT),
        W3[:, :OUT].T, W3[:, OUT:].T, b3.reshape(1, OUT),
        W4, b4.reshape(1, H),
        W5, b5.reshape(1, N),
    )
    hbm = pl.BlockSpec(memory_space=pl.ANY)
    vmem = pl.BlockSpec(memory_space=pltpu.MemorySpace.VMEM)
    return pl.pallas_call(
        _gcn_body,
        in_specs=[hbm, hbm, hbm, vmem, hbm, vmem, hbm, hbm, vmem, hbm, vmem,
                  hbm, vmem],
        out_shape=(
            jax.ShapeDtypeStruct((N, OUT), jnp.float32),
            jax.ShapeDtypeStruct((N, N), jnp.float32),
        ),
        scratch_shapes=[
            pltpu.VMEM((N, N), jnp.float32),
            pltpu.VMEM((N, N), jnp.float32),
            pltpu.VMEM((N, H), jnp.float32),
            pltpu.VMEM((H, OUT), jnp.float32),
            pltpu.VMEM((OUT, OUT), jnp.float32),
            pltpu.VMEM((H, OUT), jnp.float32),
            pltpu.VMEM((OUT, H), jnp.float32),
            pltpu.VMEM((H, N), jnp.float32),
            pltpu.SemaphoreType.DMA((8,)),
        ],
    )(*args)


# speedup vs baseline: 1.0232x; 1.0232x over previous
import jax
import jax.numpy as jnp
from jax.experimental import pallas as pl
from jax.experimental.pallas import tpu as pltpu

N = 1140
H = 600
OUT = 300


def _dot(a, b):
    return jax.lax.dot(a, b, preferred_element_type=jnp.float32)


def _bf(a):
    return a.astype(jnp.bfloat16)


def _gcn_body(x_hbm, A_hbm, W1_hbm, b1_ref, W2_hbm, b2_ref,
              W3a_hbm, W3b_hbm, b3_ref, W4_hbm, b4_ref, W5_hbm, b5_ref,
              xm_ref, out2_ref,
              x_v, A_v, W1_v, W2_v, W3a_v, W3b_v, W4_v, W5_v, sems):
    def copy(src, dst, i):
        cp = pltpu.make_async_copy(src, dst, sems.at[i])
        cp.start()
        return cp

    c_x = copy(x_hbm, x_v, 0)
    c_W1 = copy(W1_hbm, W1_v, 1)
    c_x.wait()
    c_W1.wait()
    c_A = copy(A_hbm, A_v, 2)
    y = _bf(_dot(_bf(x_v[...]), _bf(W1_v[...])))
    c_A.wait()
    c_W2 = copy(W2_hbm, W2_v, 3)
    c_W3a = copy(W3a_hbm, W3a_v, 4)
    c_W3b = copy(W3b_hbm, W3b_v, 5)
    A = _bf(A_v[...])
    x1 = _bf(jnp.maximum(_dot(A, y) + b1_ref[...], 0.0))
    A2 = _bf(_dot(A, A))
    s = jnp.sum(A_v[...], axis=1, keepdims=True)
    c_W2.wait()
    c_W3a.wait()
    c_W4 = copy(W4_hbm, W4_v, 6)
    c_W5 = copy(W5_hbm, W5_v, 7)
    W3a = _bf(W3a_v[...])
    W23a = _bf(_dot(_bf(W2_v[...]), W3a))
    c3 = _dot(_bf(b2_ref[...]), W3a) + b3_ref[...]
    u = _bf(_dot(x1, W23a))
    c_W3b.wait()
    xm = _dot(A, u) + _dot(x1, _bf(W3b_v[...])) + c3
    xm_ref[...] = xm
    P2 = _bf(_dot(A2, _bf(xm)))
    c_W4.wait()
    c_W5.wait()
    W5b = _bf(W5_v[...])
    W45 = _bf(_dot(_bf(W4_v[...]), W5b))
    b4W5 = _dot(_bf(b4_ref[...]), W5b)
    out2_ref[...] = jax.nn.sigmoid(_dot(P2, W45) + s * b4W5 + b5_ref[...])


def kernel(x, A, W1, b1, W2, b2, W3, b3, W4, b4, W5, b5):
    args = (
        x, A,
        W1, b1.reshape(1, H),
        W2, b2.reshape(1, OUT),
        W3[:, :OUT].T, W3[:, OUT:].T, b3.reshape(1, OUT),
        W4, b4.reshape(1, H),
        W5, b5.reshape(1, N),
    )
    hbm = pl.BlockSpec(memory_space=pl.ANY)
    vmem = pl.BlockSpec(memory_space=pltpu.MemorySpace.VMEM)
    return pl.pallas_call(
        _gcn_body,
        in_specs=[hbm, hbm, hbm, vmem, hbm, vmem, hbm, hbm, vmem, hbm, vmem,
                  hbm, vmem],
        out_shape=(
            jax.ShapeDtypeStruct((N, OUT), jnp.float32),
            jax.ShapeDtypeStruct((N, N), jnp.float32),
        ),
        scratch_shapes=[
            pltpu.VMEM((N, N), jnp.float32),
            pltpu.VMEM((N, N), jnp.float32),
            pltpu.VMEM((N, H), jnp.float32),
            pltpu.VMEM((H, OUT), jnp.float32),
            pltpu.VMEM((OUT, OUT), jnp.float32),
            pltpu.VMEM((H, OUT), jnp.float32),
            pltpu.VMEM((OUT, H), jnp.float32),
            pltpu.VMEM((H, N), jnp.float32),
            pltpu.SemaphoreType.DMA((8,)),
        ],
    )(*args)


# speedup vs baseline: 1.0673x; 1.0431x over previous
import jax
import jax.numpy as jnp
from jax.experimental import pallas as pl
from jax.experimental.pallas import tpu as pltpu

N = 1140
H = 600
OUT = 300


def _dot(a, b):
    return jax.lax.dot(a, b, preferred_element_type=jnp.float32)


def _bf(a):
    return a.astype(jnp.bfloat16)


def _gcn_body(x_hbm, A_hbm, W1_hbm, b1_ref, W2_hbm, b2_ref,
              W3a_hbm, W3b_hbm, b3_ref, W4_hbm, b4_ref, W5_hbm, b5_ref,
              xm_ref, out2_ref,
              x_v, A_v, W1_v, W2_v, W3a_v, W3b_v, W4_v, W5_v, sems):
    def copy(src, dst, i):
        cp = pltpu.make_async_copy(src, dst, sems.at[i])
        cp.start()
        return cp

    c_x = copy(x_hbm, x_v, 0)
    c_W1 = copy(W1_hbm, W1_v, 1)
    c_x.wait()
    c_W1.wait()
    c_A = copy(A_hbm, A_v, 2)
    y = _bf(_dot(_bf(x_v[...]), _bf(W1_v[...])))
    c_A.wait()
    c_W2 = copy(W2_hbm, W2_v, 3)
    c_W3a = copy(W3a_hbm, W3a_v, 4)
    c_W3b = copy(W3b_hbm, W3b_v, 5)
    A = _bf(A_v[...])
    x1 = _bf(jnp.maximum(_dot(A, y) + b1_ref[...], 0.0))
    s = jnp.sum(A_v[...], axis=1, keepdims=True)
    c_W2.wait()
    c_W3a.wait()
    c_W4 = copy(W4_hbm, W4_v, 6)
    c_W5 = copy(W5_hbm, W5_v, 7)
    W3a = _bf(W3a_v[...])
    W23a = _bf(_dot(_bf(W2_v[...]), W3a))
    c3 = _dot(_bf(b2_ref[...]), W3a) + b3_ref[...]
    u = _bf(_dot(x1, W23a))
    c_W3b.wait()
    xm = _dot(A, u) + _dot(x1, _bf(W3b_v[...])) + c3
    xm_ref[...] = xm
    P2 = _bf(_dot(A, _bf(_dot(A, _bf(xm)))))
    c_W4.wait()
    c_W5.wait()
    W5b = _bf(W5_v[...])
    W45 = _bf(_dot(_bf(W4_v[...]), W5b))
    b4W5 = _dot(_bf(b4_ref[...]), W5b)
    out2_ref[...] = jax.nn.sigmoid(_dot(P2, W45) + s * b4W5 + b5_ref[...])


def kernel(x, A, W1, b1, W2, b2, W3, b3, W4, b4, W5, b5):
    args = (
        x, A,
        W1, b1.reshape(1, H),
        W2, b2.reshape(1, OUT),
        W3[:, :OUT].T, W3[:, OUT:].T, b3.reshape(1, OUT),
        W4, b4.reshape(1, H),
        W5, b5.reshape(1, N),
    )
    hbm = pl.BlockSpec(memory_space=pl.ANY)
    vmem = pl.BlockSpec(memory_space=pltpu.MemorySpace.VMEM)
    return pl.pallas_call(
        _gcn_body,
        in_specs=[hbm, hbm, hbm, vmem, hbm, vmem, hbm, hbm, vmem, hbm, vmem,
                  hbm, vmem],
        out_shape=(
            jax.ShapeDtypeStruct((N, OUT), jnp.float32),
            jax.ShapeDtypeStruct((N, N), jnp.float32),
        ),
        scratch_shapes=[
            pltpu.VMEM((N, N), jnp.float32),
            pltpu.VMEM((N, N), jnp.float32),
            pltpu.VMEM((N, H), jnp.float32),
            pltpu.VMEM((H, OUT), jnp.float32),
            pltpu.VMEM((OUT, OUT), jnp.float32),
            pltpu.VMEM((H, OUT), jnp.float32),
            pltpu.VMEM((OUT, H), jnp.float32),
            pltpu.VMEM((H, N), jnp.float32),
            pltpu.SemaphoreType.DMA((8,)),
        ],
    )(*args)


# speedup vs baseline: 1.0727x; 1.0050x over previous
import jax
import jax.numpy as jnp
from jax.experimental import pallas as pl
from jax.experimental.pallas import tpu as pltpu

N = 1140
H = 600
OUT = 300


def _dot(a, b):
    return jax.lax.dot(a, b, preferred_element_type=jnp.float32)


def _bf(a):
    return a.astype(jnp.bfloat16)


def _gcn_body(x_hbm, A_hbm, W1_hbm, b1_ref, W2_hbm, b2_ref,
              W3a_hbm, W3b_hbm, b3_ref, W4_hbm, b4_ref, W5_hbm, b5_ref,
              xm_ref, out2_ref,
              x_v, A_v, W1_v, W2_v, W3a_v, W3b_v, W4_v, W5_v, sems):
    def copy(src, dst, i):
        cp = pltpu.make_async_copy(src, dst, sems.at[i])
        cp.start()
        return cp

    c_x = copy(x_hbm, x_v, 0)
    c_W1 = copy(W1_hbm, W1_v, 1)
    c_x.wait()
    c_W1.wait()
    c_A = copy(A_hbm, A_v, 2)
    y = _bf(_dot(_bf(x_v[...]), _bf(W1_v[...])))
    c_A.wait()
    c_W2 = copy(W2_hbm, W2_v, 3)
    c_W3a = copy(W3a_hbm, W3a_v, 4)
    c_W3b = copy(W3b_hbm, W3b_v, 5)
    A = _bf(A_v[...])
    x1 = _bf(jnp.maximum(_dot(A, y) + b1_ref[...], 0.0))
    s = jnp.sum(A.astype(jnp.float32), axis=1, keepdims=True)
    c_W2.wait()
    c_W3a.wait()
    c_W4 = copy(W4_hbm, W4_v, 6)
    c_W5 = copy(W5_hbm, W5_v, 7)
    W3a = _bf(W3a_v[...])
    W23a = _bf(_dot(_bf(W2_v[...]), W3a))
    c3 = _dot(_bf(b2_ref[...]), W3a) + b3_ref[...]
    u = _bf(_dot(x1, W23a))
    c_W3b.wait()
    xm = _dot(A, u) + _dot(x1, _bf(W3b_v[...])) + c3
    xm_ref[...] = xm
    P2 = _bf(_dot(A, _bf(_dot(A, _bf(xm)))))
    c_W4.wait()
    c_W5.wait()
    W5b = _bf(W5_v[...])
    W45 = _bf(_dot(_bf(W4_v[...]), W5b))
    b4W5 = _dot(_bf(b4_ref[...]), W5b)
    out2_ref[...] = jax.nn.sigmoid(_dot(P2, W45) + s * b4W5 + b5_ref[...])


def kernel(x, A, W1, b1, W2, b2, W3, b3, W4, b4, W5, b5):
    args = (
        x, A,
        W1, b1.reshape(1, H),
        W2, b2.reshape(1, OUT),
        W3[:, :OUT].T, W3[:, OUT:].T, b3.reshape(1, OUT),
        W4, b4.reshape(1, H),
        W5, b5.reshape(1, N),
    )
    hbm = pl.BlockSpec(memory_space=pl.ANY)
    vmem = pl.BlockSpec(memory_space=pltpu.MemorySpace.VMEM)
    return pl.pallas_call(
        _gcn_body,
        in_specs=[hbm, hbm, hbm, vmem, hbm, vmem, hbm, hbm, vmem, hbm, vmem,
                  hbm, vmem],
        out_shape=(
            jax.ShapeDtypeStruct((N, OUT), jnp.float32),
            jax.ShapeDtypeStruct((N, N), jnp.float32),
        ),
        scratch_shapes=[
            pltpu.VMEM((N, N), jnp.float32),
            pltpu.VMEM((N, N), jnp.float32),
            pltpu.VMEM((N, H), jnp.float32),
            pltpu.VMEM((H, OUT), jnp.float32),
            pltpu.VMEM((OUT, OUT), jnp.float32),
            pltpu.VMEM((H, OUT), jnp.float32),
            pltpu.VMEM((OUT, H), jnp.float32),
            pltpu.VMEM((H, N), jnp.float32),
            pltpu.SemaphoreType.DMA((8,)),
        ],
    )(*args)
